# baseline (speedup 1.0000x reference)
import functools

import jax
import jax.numpy as jnp
from jax import lax
from jax.experimental import pallas as pl
from jax.experimental.pallas import tpu as pltpu
from jax.experimental.pallas import tpu_sc as plsc

_NUM_CLASSES = 21
_TOP_K = 200
_PRE_NMS = 500
_CONF_THRESH = 0.05
_IOU_THRESH = 0.5
_VAR0, _VAR1 = 0.1, 0.2
_CLIP_W = _CLIP_H = 1.0
_PRE_PAD = 512
_K_PAD = 256
_NEG = -1e30

_L = 16
_NW = 32
_ROWS = 192
_ROWS_PER_W = _ROWS // _NW
_A_PAD = 20480
_NCH = _A_PAD // _L
_HI0 = 0x7F800000


def _cut_kernel(keys_hbm, out_hbm, row_v, red_v, out_v):
    wid = lax.axis_index("s") * 2 + lax.axis_index("c")

    def lane_total(cnt):
        red_v[pl.ds(0, _L)] = cnt
        red_v[pl.ds(_L, _L)] = cnt

        def rot(k, acc):
            return acc + red_v[pl.ds(k, _L)]

        return lax.fori_loop(1, _L, rot, cnt)

    def count_gt(t):
        def chunk(j, acc):
            k = row_v[pl.ds(j * _L, _L)]
            return acc + jnp.where(k > t, 1.0, 0.0)

        cnt = lax.fori_loop(0, _NCH, chunk, jnp.zeros((_L,), jnp.float32))
        return lane_total(cnt)

    def do_row(p, _):
        row = wid * _ROWS_PER_W + p
        pltpu.sync_copy(keys_hbm.at[row], row_v)

        lo = jnp.full((_L,), -1, jnp.int32)
        hi = jnp.full((_L,), _HI0, jnp.int32)

        def bis(it, carry):
            lo, hi = carry
            mid = (lo + hi) >> 1
            tot = count_gt(mid)
            pred = tot < float(_PRE_NMS)
            hi = jnp.where(pred, mid, hi)
            lo = jnp.where(pred, lo, mid)
            return lo, hi

        lo, hi = lax.fori_loop(0, 31, bis, (lo, hi))
        kcut = hi
        cgt = count_gt(kcut)

        out_v[pl.ds(0, _L)] = lax.bitcast_convert_type(kcut, jnp.float32)
        out_v[pl.ds(_L, _L)] = cgt
        pltpu.sync_copy(out_v, out_hbm.at[row])
        return 0

    lax.fori_loop(0, _ROWS_PER_W, do_row, 0)


@jax.jit
def _sc_cutoff(keys2d):
    mesh = plsc.VectorSubcoreMesh(core_axis_name="c", subcore_axis_name="s")
    fn = pl.kernel(
        _cut_kernel,
        mesh=mesh,
        out_type=[jax.ShapeDtypeStruct((_ROWS, 2 * _L), jnp.float32)],
        scratch_types=[
            pltpu.VMEM((_A_PAD,), jnp.int32),
            pltpu.VMEM((2 * _L,), jnp.float32),
            pltpu.VMEM((2 * _L,), jnp.float32),
        ],
    )
    return fn(keys2d)[0]


def _nms_kernel(cs_ref, cb_ref, outs_ref, outb_ref, ws_ref):
    n, m = cs_ref.shape
    outs_ref[...] = jnp.zeros_like(outs_ref)
    outb_ref[...] = jnp.zeros_like(outb_ref)
    ws_ref[...] = cs_ref[...]
    x1 = cb_ref[0]
    y1 = cb_ref[1]
    x2 = cb_ref[2]
    y2 = cb_ref[3]
    area = jnp.maximum(x2 - x1, 0.0) * jnp.maximum(y2 - y1, 0.0)
    iota = jax.lax.broadcasted_iota(jnp.int32, (n, m), 1)

    def body(r, carry):
        ws = ws_ref[...]
        best = jnp.max(ws, axis=1, keepdims=True)
        eq = ws == best
        pos = jnp.min(jnp.where(eq, iota, m), axis=1, keepdims=True)
        chosen = iota == pos
        chf = chosen.astype(jnp.float32)
        bx1 = jnp.sum(x1 * chf, axis=1, keepdims=True)
        by1 = jnp.sum(y1 * chf, axis=1, keepdims=True)
        bx2 = jnp.sum(x2 * chf, axis=1, keepdims=True)
        by2 = jnp.sum(y2 * chf, axis=1, keepdims=True)
        barea = jnp.sum(area * chf, axis=1, keepdims=True)
        valid = best > _CONF_THRESH
        tile = pl.multiple_of((r // 128) * 128, 128)
        off = r - tile
        wmask = jax.lax.broadcasted_iota(jnp.int32, (1, 128), 1) == off
        for ref_slice, val in (
            (outs_ref.at[:, pl.ds(tile, 128)], jnp.where(valid, best, 0.0)),
            (outb_ref.at[0, :, pl.ds(tile, 128)], jnp.where(valid, bx1, 0.0)),
            (outb_ref.at[1, :, pl.ds(tile, 128)], jnp.where(valid, by1, 0.0)),
            (outb_ref.at[2, :, pl.ds(tile, 128)], jnp.where(valid, bx2, 0.0)),
            (outb_ref.at[3, :, pl.ds(tile, 128)], jnp.where(valid, by2, 0.0)),
        ):
            ref_slice[...] = jnp.where(wmask, val, ref_slice[...])
        ltx = jnp.maximum(x1, bx1)
        lty = jnp.maximum(y1, by1)
        rbx = jnp.minimum(x2, bx2)
        rby = jnp.minimum(y2, by2)
        iw = jnp.maximum(rbx - ltx, 0.0)
        ih = jnp.maximum(rby - lty, 0.0)
        inter = iw * ih
        union = area + barea - inter
        iou = inter / jnp.maximum(union, 1e-9)
        kill = chosen | ((iou > _IOU_THRESH) & valid)
        ws_ref[...] = jnp.where(kill, _NEG, ws)
        return carry

    jax.lax.fori_loop(0, _TOP_K, body, 0)


def _run_nms(cs, cb):
    n = cs.shape[0]
    outs, outb = pl.pallas_call(
        _nms_kernel,
        out_shape=[
            jax.ShapeDtypeStruct((n, _K_PAD), jnp.float32),
            jax.ShapeDtypeStruct((4, n, _K_PAD), jnp.float32),
        ],
        scratch_shapes=[pltpu.VMEM((n, _PRE_PAD), jnp.float32)],
    )(cs, cb)
    return outs, outb


def _decode_clip(loc_delta, anchors):
    anch = anchors[None, :, :]
    cxcy = anch[..., :2] + loc_delta[..., :2] * _VAR0 * anch[..., 2:]
    wh = anch[..., 2:] * jnp.exp(loc_delta[..., 2:] * _VAR1)
    boxes = jnp.concatenate([cxcy - wh / 2.0, cxcy + wh / 2.0], axis=-1)
    x = jnp.clip(boxes[..., 0::2], 0.0, _CLIP_W)
    y = jnp.clip(boxes[..., 1::2], 0.0, _CLIP_H)
    return jnp.stack([x[..., 0], y[..., 0], x[..., 1], y[..., 1]], axis=-1)


def kernel(conf_preds, loc_delta, anchors):
    nb, na, nc = conf_preds.shape
    decoded = _decode_clip(loc_delta, anchors)
    conf_t = conf_preds.transpose(0, 2, 1)
    w = decoded[..., 2] - decoded[..., 0]
    h = decoded[..., 3] - decoded[..., 1]
    validb = (w >= 0.0) & (h >= 0.0)
    scores = jnp.maximum(jnp.where(validb[:, None, :], conf_t, 0.0), 0.0)

    n = nb * nc
    s = scores.reshape(n, na)
    keys = lax.bitcast_convert_type(s, jnp.int32)
    k2 = jnp.zeros((_ROWS, _A_PAD), jnp.int32)
    k2 = k2.at[:n, :na].set(keys)
    cut = _sc_cutoff(k2)
    kcut = lax.bitcast_convert_type(cut[:n, 0], jnp.int32)
    cgt = cut[:n, _L].astype(jnp.int32)
    m2 = _PRE_NMS - cgt

    gt = keys > kcut[:, None]
    eq = keys == kcut[:, None]
    eq_pref = jnp.cumsum(eq.astype(jnp.int32), axis=1)
    take = gt | (eq & (eq_pref <= m2[:, None]))
    rank = jnp.cumsum(take.astype(jnp.int32), axis=1) - 1
    pos = jnp.where(take, rank, _PRE_PAD - 1)
    takef = take.astype(jnp.float32)
    rows_i = jax.lax.broadcasted_iota(jnp.int32, (n, na), 0)
    anch_i = jax.lax.broadcasted_iota(jnp.int32, (n, na), 1)
    cs = jnp.zeros((n, _PRE_PAD), jnp.float32).at[rows_i, pos].add(
        s * takef, mode="drop")
    idx = jnp.zeros((n, _PRE_PAD), jnp.int32).at[rows_i, pos].add(
        anch_i * take.astype(jnp.int32), mode="drop")
    idx = idx.reshape(nb, nc, _PRE_PAD)

    cb = jnp.take_along_axis(decoded[:, None, :, :], idx[..., None], axis=2)
    cb_soa = cb.reshape(n, _PRE_PAD, 4).transpose(2, 0, 1)

    outs, outb = _run_nms(cs, cb_soa)

    out_s = outs[:, :_TOP_K].reshape(nb, nc, _TOP_K, 1)
    out_b = outb[:, :, :_TOP_K].transpose(1, 2, 0).reshape(nb, nc, _TOP_K, 4)
    return jnp.concatenate([out_s, out_b], axis=-1)


# speedup vs baseline: 1.0798x; 1.0798x over previous
import functools

import jax
import jax.numpy as jnp
from jax import lax
from jax.experimental import pallas as pl
from jax.experimental.pallas import tpu as pltpu
from jax.experimental.pallas import tpu_sc as plsc

_NUM_CLASSES = 21
_TOP_K = 200
_PRE_NMS = 500
_CONF_THRESH = 0.05
_IOU_THRESH = 0.5
_VAR0, _VAR1 = 0.1, 0.2
_CLIP_W = _CLIP_H = 1.0
_PRE_PAD = 512
_K_PAD = 256
_NEG = -1e30

_L = 16
_NW = 32
_ROWS = 192
_ROWS_PER_W = _ROWS // _NW
_A_PAD = 20480
_NCH = _A_PAD // _L
_HI0 = 0x7F800000


def _cut_kernel(keys_hbm, out_hbm, row_v, red_v, out_v):
    wid = lax.axis_index("s") * 2 + lax.axis_index("c")

    def lane_total(cnt):
        red_v[pl.ds(0, _L)] = cnt
        red_v[pl.ds(_L, _L)] = cnt

        def rot(k, acc):
            return acc + red_v[pl.ds(k, _L)]

        return lax.fori_loop(1, _L, rot, cnt)

    def count_gt(t):
        def chunk(j, acc):
            base = j * (_L * 16)
            for u in range(16):
                k = row_v[pl.ds(base + u * _L, _L)]
                acc = acc + jnp.where(k > t, 1.0, 0.0)
            return acc

        cnt = lax.fori_loop(0, _NCH // 16, chunk,
                            jnp.zeros((_L,), jnp.float32))
        return lane_total(cnt)

    def do_row(p, _):
        row = wid * _ROWS_PER_W + p
        pltpu.sync_copy(keys_hbm.at[row], row_v)

        lo = jnp.full((_L,), -1, jnp.int32)
        hi = jnp.full((_L,), _HI0, jnp.int32)

        def bis(it, carry):
            lo, hi = carry
            mid = (lo + hi) >> 1
            tot = count_gt(mid)
            pred = tot < float(_PRE_NMS)
            hi = jnp.where(pred, mid, hi)
            lo = jnp.where(pred, lo, mid)
            return lo, hi

        lo, hi = lax.fori_loop(0, 31, bis, (lo, hi))
        kcut = hi
        cgt = count_gt(kcut)

        out_v[pl.ds(0, _L)] = lax.bitcast_convert_type(kcut, jnp.float32)
        out_v[pl.ds(_L, _L)] = cgt
        pltpu.sync_copy(out_v, out_hbm.at[row])
        return 0

    lax.fori_loop(0, _ROWS_PER_W, do_row, 0)


@jax.jit
def _sc_cutoff(keys2d):
    mesh = plsc.VectorSubcoreMesh(core_axis_name="c", subcore_axis_name="s")
    fn = pl.kernel(
        _cut_kernel,
        mesh=mesh,
        out_type=[jax.ShapeDtypeStruct((_ROWS, 2 * _L), jnp.float32)],
        scratch_types=[
            pltpu.VMEM((_A_PAD,), jnp.int32),
            pltpu.VMEM((2 * _L,), jnp.float32),
            pltpu.VMEM((2 * _L,), jnp.float32),
        ],
    )
    return fn(keys2d)[0]


def _nms_kernel(cs_ref, cb_ref, outs_ref, outb_ref, ws_ref):
    n, m = cs_ref.shape
    outs_ref[...] = jnp.zeros_like(outs_ref)
    outb_ref[...] = jnp.zeros_like(outb_ref)
    ws_ref[...] = cs_ref[...]
    x1 = cb_ref[0]
    y1 = cb_ref[1]
    x2 = cb_ref[2]
    y2 = cb_ref[3]
    area = jnp.maximum(x2 - x1, 0.0) * jnp.maximum(y2 - y1, 0.0)
    iota = jax.lax.broadcasted_iota(jnp.int32, (n, m), 1)

    def body(r, carry):
        ws = ws_ref[...]
        best = jnp.max(ws, axis=1, keepdims=True)
        eq = ws == best
        pos = jnp.min(jnp.where(eq, iota, m), axis=1, keepdims=True)
        chosen = iota == pos
        chf = chosen.astype(jnp.float32)
        bx1 = jnp.sum(x1 * chf, axis=1, keepdims=True)
        by1 = jnp.sum(y1 * chf, axis=1, keepdims=True)
        bx2 = jnp.sum(x2 * chf, axis=1, keepdims=True)
        by2 = jnp.sum(y2 * chf, axis=1, keepdims=True)
        barea = jnp.sum(area * chf, axis=1, keepdims=True)
        valid = best > _CONF_THRESH
        tile = pl.multiple_of((r // 128) * 128, 128)
        off = r - tile
        wmask = jax.lax.broadcasted_iota(jnp.int32, (1, 128), 1) == off
        for ref_slice, val in (
            (outs_ref.at[:, pl.ds(tile, 128)], jnp.where(valid, best, 0.0)),
            (outb_ref.at[0, :, pl.ds(tile, 128)], jnp.where(valid, bx1, 0.0)),
            (outb_ref.at[1, :, pl.ds(tile, 128)], jnp.where(valid, by1, 0.0)),
            (outb_ref.at[2, :, pl.ds(tile, 128)], jnp.where(valid, bx2, 0.0)),
            (outb_ref.at[3, :, pl.ds(tile, 128)], jnp.where(valid, by2, 0.0)),
        ):
            ref_slice[...] = jnp.where(wmask, val, ref_slice[...])
        ltx = jnp.maximum(x1, bx1)
        lty = jnp.maximum(y1, by1)
        rbx = jnp.minimum(x2, bx2)
        rby = jnp.minimum(y2, by2)
        iw = jnp.maximum(rbx - ltx, 0.0)
        ih = jnp.maximum(rby - lty, 0.0)
        inter = iw * ih
        union = area + barea - inter
        iou = inter / jnp.maximum(union, 1e-9)
        kill = chosen | ((iou > _IOU_THRESH) & valid)
        ws_ref[...] = jnp.where(kill, _NEG, ws)
        return carry

    jax.lax.fori_loop(0, _TOP_K, body, 0)


def _run_nms(cs, cb):
    n = cs.shape[0]
    outs, outb = pl.pallas_call(
        _nms_kernel,
        out_shape=[
            jax.ShapeDtypeStruct((n, _K_PAD), jnp.float32),
            jax.ShapeDtypeStruct((4, n, _K_PAD), jnp.float32),
        ],
        scratch_shapes=[pltpu.VMEM((n, _PRE_PAD), jnp.float32)],
    )(cs, cb)
    return outs, outb


def _decode_clip(loc_delta, anchors):
    anch = anchors[None, :, :]
    cxcy = anch[..., :2] + loc_delta[..., :2] * _VAR0 * anch[..., 2:]
    wh = anch[..., 2:] * jnp.exp(loc_delta[..., 2:] * _VAR1)
    boxes = jnp.concatenate([cxcy - wh / 2.0, cxcy + wh / 2.0], axis=-1)
    x = jnp.clip(boxes[..., 0::2], 0.0, _CLIP_W)
    y = jnp.clip(boxes[..., 1::2], 0.0, _CLIP_H)
    return jnp.stack([x[..., 0], y[..., 0], x[..., 1], y[..., 1]], axis=-1)


def kernel(conf_preds, loc_delta, anchors):
    nb, na, nc = conf_preds.shape
    decoded = _decode_clip(loc_delta, anchors)
    conf_t = conf_preds.transpose(0, 2, 1)
    w = decoded[..., 2] - decoded[..., 0]
    h = decoded[..., 3] - decoded[..., 1]
    validb = (w >= 0.0) & (h >= 0.0)
    scores = jnp.maximum(jnp.where(validb[:, None, :], conf_t, 0.0), 0.0)

    n = nb * nc
    s = scores.reshape(n, na)
    keys = lax.bitcast_convert_type(s, jnp.int32)
    k2 = jnp.zeros((_ROWS, _A_PAD), jnp.int32)
    k2 = k2.at[:n, :na].set(keys)
    cut = _sc_cutoff(k2)
    kcut = lax.bitcast_convert_type(cut[:n, 0], jnp.int32)
    cgt = cut[:n, _L].astype(jnp.int32)
    m2 = _PRE_NMS - cgt

    gt = keys > kcut[:, None]
    eq = keys == kcut[:, None]
    eq_pref = jnp.cumsum(eq.astype(jnp.int32), axis=1)
    take = gt | (eq & (eq_pref <= m2[:, None]))
    rank = jnp.cumsum(take.astype(jnp.int32), axis=1) - 1
    pos = jnp.where(take, rank, _PRE_PAD - 1)
    takef = take.astype(jnp.float32)
    rows_i = jax.lax.broadcasted_iota(jnp.int32, (n, na), 0)
    anch_i = jax.lax.broadcasted_iota(jnp.int32, (n, na), 1)
    cs = jnp.zeros((n, _PRE_PAD), jnp.float32).at[rows_i, pos].add(
        s * takef, mode="drop")
    idx = jnp.zeros((n, _PRE_PAD), jnp.int32).at[rows_i, pos].add(
        anch_i * take.astype(jnp.int32), mode="drop")
    idx = idx.reshape(nb, nc, _PRE_PAD)

    cb = jnp.take_along_axis(decoded[:, None, :, :], idx[..., None], axis=2)
    cb_soa = cb.reshape(n, _PRE_PAD, 4).transpose(2, 0, 1)

    outs, outb = _run_nms(cs, cb_soa)

    out_s = outs[:, :_TOP_K].reshape(nb, nc, _TOP_K, 1)
    out_b = outb[:, :, :_TOP_K].transpose(1, 2, 0).reshape(nb, nc, _TOP_K, 4)
    return jnp.concatenate([out_s, out_b], axis=-1)


# speedup vs baseline: 6.9155x; 6.4043x over previous
import functools

import jax
import jax.numpy as jnp
from jax import lax
from jax.experimental import pallas as pl
from jax.experimental.pallas import tpu as pltpu
from jax.experimental.pallas import tpu_sc as plsc

_NUM_CLASSES = 21
_TOP_K = 200
_PRE_NMS = 500
_CONF_THRESH = 0.05
_IOU_THRESH = 0.5
_VAR0, _VAR1 = 0.1, 0.2
_CLIP_W = _CLIP_H = 1.0
_PRE_PAD = 512
_K_PAD = 256
_NEG = -1e30

_L = 16
_NW = 32
_ROWS = 192
_ROWS_PER_W = _ROWS // _NW
_A_PAD = 20480
_NCH = _A_PAD // _L
_HI0 = 0x7F800000


def _cut_kernel(keys_hbm, out_hbm, row_v, red_v, out_v):
    wid = lax.axis_index("s") * 2 + lax.axis_index("c")

    def lane_total(cnt):
        red_v[pl.ds(0, _L)] = cnt
        red_v[pl.ds(_L, _L)] = cnt

        def rot(k, acc):
            return acc + red_v[pl.ds(k, _L)]

        return lax.fori_loop(1, _L, rot, cnt)

    def count_gt(t):
        def chunk(j, acc):
            base = j * (_L * 16)
            for u in range(16):
                k = row_v[pl.ds(base + u * _L, _L)]
                acc = acc + jnp.where(k > t, 1.0, 0.0)
            return acc

        cnt = lax.fori_loop(0, _NCH // 16, chunk,
                            jnp.zeros((_L,), jnp.float32))
        return lane_total(cnt)

    def do_row(p, _):
        row = wid * _ROWS_PER_W + p
        pltpu.sync_copy(keys_hbm.at[row], row_v)

        lo = jnp.full((_L,), -1, jnp.int32)
        hi = jnp.full((_L,), _HI0, jnp.int32)

        def bis(it, carry):
            lo, hi = carry
            mid = (lo + hi) >> 1
            tot = count_gt(mid)
            pred = tot < float(_PRE_NMS)
            hi = jnp.where(pred, mid, hi)
            lo = jnp.where(pred, lo, mid)
            return lo, hi

        lo, hi = lax.fori_loop(0, 31, bis, (lo, hi))
        kcut = hi
        cgt = count_gt(kcut)

        out_v[pl.ds(0, _L)] = lax.bitcast_convert_type(kcut, jnp.float32)
        out_v[pl.ds(_L, _L)] = cgt
        pltpu.sync_copy(out_v, out_hbm.at[row])
        return 0

    lax.fori_loop(0, _ROWS_PER_W, do_row, 0)


@jax.jit
def _sc_cutoff(keys2d):
    mesh = plsc.VectorSubcoreMesh(core_axis_name="c", subcore_axis_name="s")
    fn = pl.kernel(
        _cut_kernel,
        mesh=mesh,
        out_type=[jax.ShapeDtypeStruct((_ROWS, 2 * _L), jnp.float32)],
        scratch_types=[
            pltpu.VMEM((_A_PAD,), jnp.int32),
            pltpu.VMEM((2 * _L,), jnp.float32),
            pltpu.VMEM((2 * _L,), jnp.float32),
        ],
    )
    return fn(keys2d)[0]


def _nms_kernel(cs_ref, cb_ref, outs_ref, outb_ref, ws_ref):
    n, m = cs_ref.shape
    outs_ref[...] = jnp.zeros_like(outs_ref)
    outb_ref[...] = jnp.zeros_like(outb_ref)
    ws_ref[...] = cs_ref[...]
    x1 = cb_ref[0]
    y1 = cb_ref[1]
    x2 = cb_ref[2]
    y2 = cb_ref[3]
    area = jnp.maximum(x2 - x1, 0.0) * jnp.maximum(y2 - y1, 0.0)
    iota = jax.lax.broadcasted_iota(jnp.int32, (n, m), 1)

    def body(r, carry):
        ws = ws_ref[...]
        best = jnp.max(ws, axis=1, keepdims=True)
        eq = ws == best
        pos = jnp.min(jnp.where(eq, iota, m), axis=1, keepdims=True)
        chosen = iota == pos
        chf = chosen.astype(jnp.float32)
        bx1 = jnp.sum(x1 * chf, axis=1, keepdims=True)
        by1 = jnp.sum(y1 * chf, axis=1, keepdims=True)
        bx2 = jnp.sum(x2 * chf, axis=1, keepdims=True)
        by2 = jnp.sum(y2 * chf, axis=1, keepdims=True)
        barea = jnp.sum(area * chf, axis=1, keepdims=True)
        valid = best > _CONF_THRESH
        tile = pl.multiple_of((r // 128) * 128, 128)
        off = r - tile
        wmask = jax.lax.broadcasted_iota(jnp.int32, (1, 128), 1) == off
        for ref_slice, val in (
            (outs_ref.at[:, pl.ds(tile, 128)], jnp.where(valid, best, 0.0)),
            (outb_ref.at[0, :, pl.ds(tile, 128)], jnp.where(valid, bx1, 0.0)),
            (outb_ref.at[1, :, pl.ds(tile, 128)], jnp.where(valid, by1, 0.0)),
            (outb_ref.at[2, :, pl.ds(tile, 128)], jnp.where(valid, bx2, 0.0)),
            (outb_ref.at[3, :, pl.ds(tile, 128)], jnp.where(valid, by2, 0.0)),
        ):
            ref_slice[...] = jnp.where(wmask, val, ref_slice[...])
        ltx = jnp.maximum(x1, bx1)
        lty = jnp.maximum(y1, by1)
        rbx = jnp.minimum(x2, bx2)
        rby = jnp.minimum(y2, by2)
        iw = jnp.maximum(rbx - ltx, 0.0)
        ih = jnp.maximum(rby - lty, 0.0)
        inter = iw * ih
        union = area + barea - inter
        iou = inter / jnp.maximum(union, 1e-9)
        kill = chosen | ((iou > _IOU_THRESH) & valid)
        ws_ref[...] = jnp.where(kill, _NEG, ws)
        return carry

    jax.lax.fori_loop(0, _TOP_K, body, 0)


def _run_nms(cs, cb):
    n = cs.shape[0]
    outs, outb = pl.pallas_call(
        _nms_kernel,
        out_shape=[
            jax.ShapeDtypeStruct((n, _K_PAD), jnp.float32),
            jax.ShapeDtypeStruct((4, n, _K_PAD), jnp.float32),
        ],
        scratch_shapes=[pltpu.VMEM((n, _PRE_PAD), jnp.float32)],
    )(cs, cb)
    return outs, outb


def _decode_clip(loc_delta, anchors):
    anch = anchors[None, :, :]
    cxcy = anch[..., :2] + loc_delta[..., :2] * _VAR0 * anch[..., 2:]
    wh = anch[..., 2:] * jnp.exp(loc_delta[..., 2:] * _VAR1)
    boxes = jnp.concatenate([cxcy - wh / 2.0, cxcy + wh / 2.0], axis=-1)
    x = jnp.clip(boxes[..., 0::2], 0.0, _CLIP_W)
    y = jnp.clip(boxes[..., 1::2], 0.0, _CLIP_H)
    return jnp.stack([x[..., 0], y[..., 0], x[..., 1], y[..., 1]], axis=-1)


def kernel(conf_preds, loc_delta, anchors):
    nb, na, nc = conf_preds.shape
    decoded = _decode_clip(loc_delta, anchors)
    conf_t = conf_preds.transpose(0, 2, 1)
    w = decoded[..., 2] - decoded[..., 0]
    h = decoded[..., 3] - decoded[..., 1]
    validb = (w >= 0.0) & (h >= 0.0)
    scores = jnp.maximum(jnp.where(validb[:, None, :], conf_t, 0.0), 0.0)

    n = nb * nc
    s = scores.reshape(n, na)
    keys = lax.bitcast_convert_type(s, jnp.int32)
    k2 = jnp.zeros((_ROWS, _A_PAD), jnp.int32)
    k2 = k2.at[:n, :na].set(keys)
    cut = _sc_cutoff(k2)
    kcut = lax.bitcast_convert_type(cut[:n, 0], jnp.int32)
    cgt = cut[:n, _L].astype(jnp.int32)
    m2 = _PRE_NMS - cgt

    gt = keys > kcut[:, None]
    eq = keys == kcut[:, None]
    packed = jnp.where(gt, 1, 0) + jnp.where(eq, 1 << 15, 0)
    cpk = jnp.cumsum(packed, axis=1)
    gt_pref = cpk & 0x7FFF
    eq_pref = cpk >> 15
    c_incl = gt_pref + jnp.minimum(eq_pref, m2[:, None])

    jq = jax.lax.broadcasted_iota(jnp.int32, (n, _PRE_PAD), 1) + 1
    lo = jnp.full((n, _PRE_PAD), -1, jnp.int32)
    hi = jnp.full((n, _PRE_PAD), na, jnp.int32)
    for _ in range(15):
        mid = (lo + hi) >> 1
        v = jnp.take_along_axis(c_incl, jnp.clip(mid, 0, na - 1), axis=1)
        pred = (v >= jq) & (mid >= 0)
        hi = jnp.where(pred, mid, hi)
        lo = jnp.where(pred, lo, mid)
    slot_ok = jq <= _PRE_NMS
    src = jnp.where(slot_ok, jnp.minimum(hi, na - 1), 0)
    cs = jnp.where(slot_ok, jnp.take_along_axis(s, src, axis=1), 0.0)
    idx = src.reshape(nb, nc, _PRE_PAD)

    cb = jnp.take_along_axis(decoded[:, None, :, :], idx[..., None], axis=2)
    cb_soa = cb.reshape(n, _PRE_PAD, 4).transpose(2, 0, 1)

    outs, outb = _run_nms(cs, cb_soa)

    out_s = outs[:, :_TOP_K].reshape(nb, nc, _TOP_K, 1)
    out_b = outb[:, :, :_TOP_K].transpose(1, 2, 0).reshape(nb, nc, _TOP_K, 4)
    return jnp.concatenate([out_s, out_b], axis=-1)
